# baseline (speedup 1.0000x reference)
import functools

import jax
import jax.numpy as jnp
from jax import lax
from jax.experimental import pallas as pl
from jax.experimental.pallas import tpu as pltpu
from jax.experimental.pallas import tpu_sc as plsc

DIM = 128
BATCH = 4096
N_FIELDS = 26
N_BIG = 6
SMALL_VOCAB = 1000


def _build():
    info = plsc.get_sparse_core_info()
    nc, ns = info.num_cores, info.num_subcores
    nw = nc * ns
    bpw = BATCH // nw

    depth = 3
    chunk = 128
    n_chunks = bpw // chunk
    n_staged = 9
    staged_fields = list(range(N_BIG, N_BIG + n_staged))
    hbm_fields = [f for f in range(N_FIELDS) if f not in staged_fields]
    lead = 6
    order = hbm_fields[:lead]
    rest_h, rest_s = hbm_fields[lead:], list(staged_fields)
    while rest_h or rest_s:
        if rest_s:
            order.append(rest_s.pop(0))
        if rest_h:
            order.append(rest_h.pop(0))
    items = [(f, c) for f in order for c in range(n_chunks)]
    first_staged_item = next(
        i for i, (f, _) in enumerate(items) if f in staged_fields
    )

    mesh = plsc.VectorSubcoreMesh(core_axis_name="c", subcore_axis_name="s")
    out_type = tuple(
        jax.ShapeDtypeStruct((BATCH, DIM), jnp.float32) for _ in range(N_FIELDS)
    )

    @functools.partial(
        pl.kernel,
        mesh=mesh,
        out_type=out_type,
        scratch_types=[
            pltpu.VMEM((N_FIELDS, bpw), jnp.int32),
            *[pltpu.VMEM((chunk, DIM), jnp.float32) for _ in range(depth)],
            *[
                pltpu.VMEM_SHARED((SMALL_VOCAB, DIM), jnp.float32)
                for _ in range(n_staged)
            ],
            pltpu.SemaphoreType.DMA,
            *[pltpu.SemaphoreType.DMA for _ in range(depth)],
            *[pltpu.SemaphoreType.DMA for _ in range(depth)],
        ],
    )
    def emb_kernel(*refs):
        idx_stack = refs[0]
        tables = refs[1 : 1 + N_FIELDS]
        outs = refs[1 + N_FIELDS : 1 + 2 * N_FIELDS]
        scratch = refs[1 + 2 * N_FIELDS :]
        idx_v = scratch[0]
        rows = scratch[1 : 1 + depth]
        shared = scratch[1 + depth : 1 + depth + n_staged]
        isem = scratch[1 + depth + n_staged]
        gsems = scratch[2 + depth + n_staged : 2 + 2 * depth + n_staged]
        ssems = scratch[2 + 2 * depth + n_staged :]

        sid = lax.axis_index("s")
        wid = sid * nc + lax.axis_index("c")
        base = wid * bpw

        icp = pltpu.async_copy(
            idx_stack.at[:, pl.ds(base, bpw)], idx_v, isem
        )

        for t in range(n_staged):

            @pl.when(sid == t)
            def _(t=t):
                pltpu.sync_copy(tables[staged_fields[t]], shared[t])

        icp.wait()

        def fire_gather(pos):
            f, c = items[pos]
            b = pos % depth
            idx = idx_v.at[f, pl.ds(c * chunk, chunk)]
            src = (
                shared[f - N_BIG].at[idx]
                if f in staged_fields
                else tables[f].at[idx]
            )
            return pltpu.async_copy(src, rows[b], gsems[b])

        def fire_store(pos):
            f, c = items[pos]
            b = pos % depth
            return pltpu.async_copy(
                rows[b], outs[f].at[pl.ds(base + c * chunk, chunk)], ssems[b]
            )

        n_items = len(items)
        gcps = [None] * n_items
        scps = [None] * n_items
        for j in range(min(depth - 1, n_items)):
            gcps[j] = fire_gather(j)
        for i in range(n_items):
            if i >= 1:
                scps[i - 1].wait()
            j = i + depth - 1
            if j < n_items:
                if j == first_staged_item:
                    plsc.subcore_barrier()
                gcps[j] = fire_gather(j)
            gcps[i].wait()
            scps[i] = fire_store(i)
        scps[n_items - 1].wait()

    return emb_kernel


_emb_kernel = _build()


def kernel(
    feat_0, feat_1, feat_2, feat_3, feat_4, feat_5, feat_6, feat_7,
    feat_8, feat_9, feat_10, feat_11, feat_12, feat_13, feat_14, feat_15,
    feat_16, feat_17, feat_18, feat_19, feat_20, feat_21, feat_22, feat_23,
    feat_24, feat_25,
    W_0, W_1, W_2, W_3, W_4, W_5, W_6, W_7,
    W_8, W_9, W_10, W_11, W_12, W_13, W_14, W_15,
    W_16, W_17, W_18, W_19, W_20, W_21, W_22, W_23,
    W_24, W_25,
):
    args = locals()
    feats = [args[f"feat_{i}"] for i in range(N_FIELDS)]
    tables = [args[f"W_{i}"] for i in range(N_FIELDS)]
    idx_stack = jnp.stack(feats)
    outs = _emb_kernel(idx_stack, *tables)
    return tuple(o.reshape(BATCH, 1, DIM) for o in outs)


# speedup vs baseline: 1.0387x; 1.0387x over previous
import functools

import jax
import jax.numpy as jnp
from jax import lax
from jax.experimental import pallas as pl
from jax.experimental.pallas import tpu as pltpu
from jax.experimental.pallas import tpu_sc as plsc

DIM = 128
BATCH = 4096
N_FIELDS = 26
N_BIG = 6
SMALL_VOCAB = 1000


def _build():
    info = plsc.get_sparse_core_info()
    nc, ns = info.num_cores, info.num_subcores
    nw = nc * ns
    bpw = BATCH // nw

    depth = 3
    chunk = 128
    n_chunks = bpw // chunk
    n_staged = 9
    staged_fields = list(range(N_BIG, N_BIG + n_staged))
    hbm_fields = [f for f in range(N_FIELDS) if f not in staged_fields]
    order = hbm_fields + staged_fields
    items = [(f, c) for f in order for c in range(n_chunks)]
    first_staged_item = next(
        i for i, (f, _) in enumerate(items) if f in staged_fields
    )

    mesh = plsc.VectorSubcoreMesh(core_axis_name="c", subcore_axis_name="s")
    out_type = tuple(
        jax.ShapeDtypeStruct((BATCH, DIM), jnp.float32) for _ in range(N_FIELDS)
    )

    @functools.partial(
        pl.kernel,
        mesh=mesh,
        out_type=out_type,
        scratch_types=[
            pltpu.VMEM((N_FIELDS, bpw), jnp.int32),
            *[pltpu.VMEM((chunk, DIM), jnp.float32) for _ in range(depth)],
            *[
                pltpu.VMEM_SHARED((SMALL_VOCAB, DIM), jnp.float32)
                for _ in range(n_staged)
            ],
            pltpu.SemaphoreType.DMA,
            *[pltpu.SemaphoreType.DMA for _ in range(depth)],
            *[pltpu.SemaphoreType.DMA for _ in range(depth)],
        ],
    )
    def emb_kernel(*refs):
        idx_stack = refs[0]
        tables = refs[1 : 1 + N_FIELDS]
        outs = refs[1 + N_FIELDS : 1 + 2 * N_FIELDS]
        scratch = refs[1 + 2 * N_FIELDS :]
        idx_v = scratch[0]
        rows = scratch[1 : 1 + depth]
        shared = scratch[1 + depth : 1 + depth + n_staged]
        isem = scratch[1 + depth + n_staged]
        gsems = scratch[2 + depth + n_staged : 2 + 2 * depth + n_staged]
        ssems = scratch[2 + 2 * depth + n_staged :]

        sid = lax.axis_index("s")
        wid = sid * nc + lax.axis_index("c")
        base = wid * bpw

        icp = pltpu.async_copy(
            idx_stack.at[:, pl.ds(base, bpw)], idx_v, isem
        )

        for t in range(n_staged):

            @pl.when(sid == t)
            def _(t=t):
                pltpu.sync_copy(tables[staged_fields[t]], shared[t])

        icp.wait()

        def fire_gather(pos):
            f, c = items[pos]
            b = pos % depth
            idx = idx_v.at[f, pl.ds(c * chunk, chunk)]
            src = (
                shared[f - N_BIG].at[idx]
                if f in staged_fields
                else tables[f].at[idx]
            )
            return pltpu.async_copy(src, rows[b], gsems[b])

        def fire_store(pos):
            f, c = items[pos]
            b = pos % depth
            return pltpu.async_copy(
                rows[b], outs[f].at[pl.ds(base + c * chunk, chunk)], ssems[b]
            )

        n_items = len(items)
        gcps = [None] * n_items
        scps = [None] * n_items
        for j in range(min(depth - 1, n_items)):
            gcps[j] = fire_gather(j)
        for i in range(n_items):
            if i >= 1:
                scps[i - 1].wait()
            j = i + depth - 1
            if j < n_items:
                if j == first_staged_item:
                    plsc.subcore_barrier()
                gcps[j] = fire_gather(j)
            gcps[i].wait()
            scps[i] = fire_store(i)
        scps[n_items - 1].wait()

    return emb_kernel


_emb_kernel = _build()


def kernel(
    feat_0, feat_1, feat_2, feat_3, feat_4, feat_5, feat_6, feat_7,
    feat_8, feat_9, feat_10, feat_11, feat_12, feat_13, feat_14, feat_15,
    feat_16, feat_17, feat_18, feat_19, feat_20, feat_21, feat_22, feat_23,
    feat_24, feat_25,
    W_0, W_1, W_2, W_3, W_4, W_5, W_6, W_7,
    W_8, W_9, W_10, W_11, W_12, W_13, W_14, W_15,
    W_16, W_17, W_18, W_19, W_20, W_21, W_22, W_23,
    W_24, W_25,
):
    args = locals()
    feats = [args[f"feat_{i}"] for i in range(N_FIELDS)]
    tables = [args[f"W_{i}"] for i in range(N_FIELDS)]
    idx_stack = jnp.stack(feats)
    outs = _emb_kernel(idx_stack, *tables)
    return tuple(o.reshape(BATCH, 1, DIM) for o in outs)


# speedup vs baseline: 1.0416x; 1.0028x over previous
import functools

import jax
import jax.numpy as jnp
from jax import lax
from jax.experimental import pallas as pl
from jax.experimental.pallas import tpu as pltpu
from jax.experimental.pallas import tpu_sc as plsc

DIM = 128
BATCH = 4096
N_FIELDS = 26
N_BIG = 6
SMALL_VOCAB = 1000


def _build():
    info = plsc.get_sparse_core_info()
    nc, ns = info.num_cores, info.num_subcores
    nw = nc * ns
    bpw = BATCH // nw

    depth = 3
    chunk = 128
    n_chunks = bpw // chunk
    n_staged = 9
    staged_fields = list(range(N_BIG, N_BIG + n_staged))
    hbm_fields = [f for f in range(N_FIELDS) if f not in staged_fields]
    order = hbm_fields + staged_fields
    items = [(f, c) for f in order for c in range(n_chunks)]
    first_staged_item = next(
        i for i, (f, _) in enumerate(items) if f in staged_fields
    )

    mesh = plsc.VectorSubcoreMesh(core_axis_name="c", subcore_axis_name="s")
    out_type = tuple(
        jax.ShapeDtypeStruct((BATCH, DIM), jnp.float32) for _ in range(N_FIELDS)
    )

    @functools.partial(
        pl.kernel,
        mesh=mesh,
        out_type=out_type,
        scratch_types=[
            pltpu.VMEM((N_FIELDS, bpw), jnp.int32),
            *[pltpu.VMEM((chunk, DIM), jnp.float32) for _ in range(depth)],
            *[
                pltpu.VMEM_SHARED((SMALL_VOCAB, DIM), jnp.float32)
                for _ in range(n_staged)
            ],
            pltpu.SemaphoreType.DMA,
            *[pltpu.SemaphoreType.DMA for _ in range(depth)],
            *[pltpu.SemaphoreType.DMA for _ in range(depth)],
        ],
    )
    def emb_kernel(*refs):
        idx_stack = refs[0]
        tables = refs[1 : 1 + N_FIELDS]
        outs = refs[1 + N_FIELDS : 1 + 2 * N_FIELDS]
        scratch = refs[1 + 2 * N_FIELDS :]
        idx_v = scratch[0]
        rows = scratch[1 : 1 + depth]
        shared = scratch[1 + depth : 1 + depth + n_staged]
        isem = scratch[1 + depth + n_staged]
        gsems = scratch[2 + depth + n_staged : 2 + 2 * depth + n_staged]
        ssems = scratch[2 + 2 * depth + n_staged :]

        sid = lax.axis_index("s")
        wid = sid * nc + lax.axis_index("c")
        base = wid * bpw

        icps = [
            pltpu.async_copy(
                idx_stack.at[i, pl.ds(base, bpw)], idx_v.at[i], isem
            )
            for i in range(N_FIELDS)
        ]

        for t in range(n_staged):

            @pl.when(sid == t)
            def _(t=t):
                pltpu.sync_copy(tables[staged_fields[t]], shared[t])

        for cp in icps:
            cp.wait()

        def fire_gather(pos):
            f, c = items[pos]
            b = pos % depth
            idx = idx_v.at[f, pl.ds(c * chunk, chunk)]
            src = (
                shared[f - N_BIG].at[idx]
                if f in staged_fields
                else tables[f].at[idx]
            )
            return pltpu.async_copy(src, rows[b], gsems[b])

        def fire_store(pos):
            f, c = items[pos]
            b = pos % depth
            return pltpu.async_copy(
                rows[b], outs[f].at[pl.ds(base + c * chunk, chunk)], ssems[b]
            )

        n_items = len(items)
        gcps = [None] * n_items
        scps = [None] * n_items
        for j in range(min(depth - 1, n_items)):
            gcps[j] = fire_gather(j)
        for i in range(n_items):
            if i >= 1:
                scps[i - 1].wait()
            j = i + depth - 1
            if j < n_items:
                if j == first_staged_item:
                    plsc.subcore_barrier()
                gcps[j] = fire_gather(j)
            gcps[i].wait()
            scps[i] = fire_store(i)
        scps[n_items - 1].wait()

    return emb_kernel


_emb_kernel = _build()


def kernel(
    feat_0, feat_1, feat_2, feat_3, feat_4, feat_5, feat_6, feat_7,
    feat_8, feat_9, feat_10, feat_11, feat_12, feat_13, feat_14, feat_15,
    feat_16, feat_17, feat_18, feat_19, feat_20, feat_21, feat_22, feat_23,
    feat_24, feat_25,
    W_0, W_1, W_2, W_3, W_4, W_5, W_6, W_7,
    W_8, W_9, W_10, W_11, W_12, W_13, W_14, W_15,
    W_16, W_17, W_18, W_19, W_20, W_21, W_22, W_23,
    W_24, W_25,
):
    args = locals()
    feats = [args[f"feat_{i}"] for i in range(N_FIELDS)]
    tables = [args[f"W_{i}"] for i in range(N_FIELDS)]
    idx_stack = jnp.stack(feats)
    outs = _emb_kernel(idx_stack, *tables)
    return tuple(o.reshape(BATCH, 1, DIM) for o in outs)


# speedup vs baseline: 1.0497x; 1.0078x over previous
import functools

import jax
import jax.numpy as jnp
from jax import lax
from jax.experimental import pallas as pl
from jax.experimental.pallas import tpu as pltpu
from jax.experimental.pallas import tpu_sc as plsc

DIM = 128
BATCH = 4096
N_FIELDS = 26
N_BIG = 6
SMALL_VOCAB = 1000


def _build():
    info = plsc.get_sparse_core_info()
    nc, ns = info.num_cores, info.num_subcores
    nw = nc * ns
    bpw = BATCH // nw

    depth = 3
    chunk = 128
    n_chunks = bpw // chunk
    n_staged = 9
    staged_fields = list(range(N_BIG, N_BIG + n_staged))
    hbm_fields = [f for f in range(N_FIELDS) if f not in staged_fields]
    order = hbm_fields + staged_fields
    items = [(f, c) for f in order for c in range(n_chunks)]
    first_staged_item = next(
        i for i, (f, _) in enumerate(items) if f in staged_fields
    )

    mesh = plsc.VectorSubcoreMesh(core_axis_name="c", subcore_axis_name="s")
    out_type = tuple(
        jax.ShapeDtypeStruct((BATCH, DIM), jnp.float32) for _ in range(N_FIELDS)
    )

    @functools.partial(
        pl.kernel,
        mesh=mesh,
        out_type=out_type,
        scratch_types=[
            pltpu.VMEM((N_FIELDS, bpw), jnp.int32),
            *[pltpu.VMEM((chunk, DIM), jnp.float32) for _ in range(depth)],
            *[
                pltpu.VMEM_SHARED((SMALL_VOCAB, DIM), jnp.float32)
                for _ in range(n_staged)
            ],
            pltpu.SemaphoreType.DMA,
            *[pltpu.SemaphoreType.DMA for _ in range(depth)],
            *[pltpu.SemaphoreType.DMA for _ in range(depth)],
        ],
    )
    def emb_kernel(*refs):
        feats = refs[:N_FIELDS]
        tables = refs[N_FIELDS : 2 * N_FIELDS]
        outs = refs[2 * N_FIELDS : 3 * N_FIELDS]
        scratch = refs[3 * N_FIELDS :]
        idx_v = scratch[0]
        rows = scratch[1 : 1 + depth]
        shared = scratch[1 + depth : 1 + depth + n_staged]
        isem = scratch[1 + depth + n_staged]
        gsems = scratch[2 + depth + n_staged : 2 + 2 * depth + n_staged]
        ssems = scratch[2 + 2 * depth + n_staged :]

        sid = lax.axis_index("s")
        wid = sid * nc + lax.axis_index("c")
        base = wid * bpw

        icps = [
            pltpu.async_copy(feats[i].at[pl.ds(base, bpw)], idx_v.at[i], isem)
            for i in range(N_FIELDS)
        ]

        for t in range(n_staged):

            @pl.when(sid == t)
            def _(t=t):
                pltpu.sync_copy(tables[staged_fields[t]], shared[t])

        for cp in icps:
            cp.wait()

        def fire_gather(pos):
            f, c = items[pos]
            b = pos % depth
            idx = idx_v.at[f, pl.ds(c * chunk, chunk)]
            src = (
                shared[f - N_BIG].at[idx]
                if f in staged_fields
                else tables[f].at[idx]
            )
            return pltpu.async_copy(src, rows[b], gsems[b])

        def fire_store(pos):
            f, c = items[pos]
            b = pos % depth
            return pltpu.async_copy(
                rows[b], outs[f].at[pl.ds(base + c * chunk, chunk)], ssems[b]
            )

        n_items = len(items)
        gcps = [None] * n_items
        scps = [None] * n_items
        for j in range(min(depth - 1, n_items)):
            gcps[j] = fire_gather(j)
        for i in range(n_items):
            if i >= 1:
                scps[i - 1].wait()
            j = i + depth - 1
            if j < n_items:
                if j == first_staged_item:
                    plsc.subcore_barrier()
                gcps[j] = fire_gather(j)
            gcps[i].wait()
            scps[i] = fire_store(i)
        scps[n_items - 1].wait()

    return emb_kernel


_emb_kernel = _build()


def kernel(
    feat_0, feat_1, feat_2, feat_3, feat_4, feat_5, feat_6, feat_7,
    feat_8, feat_9, feat_10, feat_11, feat_12, feat_13, feat_14, feat_15,
    feat_16, feat_17, feat_18, feat_19, feat_20, feat_21, feat_22, feat_23,
    feat_24, feat_25,
    W_0, W_1, W_2, W_3, W_4, W_5, W_6, W_7,
    W_8, W_9, W_10, W_11, W_12, W_13, W_14, W_15,
    W_16, W_17, W_18, W_19, W_20, W_21, W_22, W_23,
    W_24, W_25,
):
    args = locals()
    feats = [args[f"feat_{i}"] for i in range(N_FIELDS)]
    tables = [args[f"W_{i}"] for i in range(N_FIELDS)]
    outs = _emb_kernel(*feats, *tables)
    return tuple(o.reshape(BATCH, 1, DIM) for o in outs)


# speedup vs baseline: 1.0555x; 1.0055x over previous
import functools

import jax
import jax.numpy as jnp
from jax import lax
from jax.experimental import pallas as pl
from jax.experimental.pallas import tpu as pltpu
from jax.experimental.pallas import tpu_sc as plsc

DIM = 128
BATCH = 4096
N_FIELDS = 26
N_BIG = 6
SMALL_VOCAB = 1000


def _build():
    info = plsc.get_sparse_core_info()
    nc, ns = info.num_cores, info.num_subcores
    nw = nc * ns
    bpw = BATCH // nw

    depth = 4
    chunk = 128
    n_chunks = bpw // chunk
    n_staged = 7
    staged_fields = list(range(N_BIG, N_BIG + n_staged))
    hbm_fields = [f for f in range(N_FIELDS) if f not in staged_fields]
    order = hbm_fields + staged_fields
    items = [(f, c) for f in order for c in range(n_chunks)]
    first_staged_item = next(
        i for i, (f, _) in enumerate(items) if f in staged_fields
    )

    mesh = plsc.VectorSubcoreMesh(core_axis_name="c", subcore_axis_name="s")
    out_type = tuple(
        jax.ShapeDtypeStruct((BATCH, DIM), jnp.float32) for _ in range(N_FIELDS)
    )

    @functools.partial(
        pl.kernel,
        mesh=mesh,
        out_type=out_type,
        scratch_types=[
            pltpu.VMEM((N_FIELDS, bpw), jnp.int32),
            *[pltpu.VMEM((chunk, DIM), jnp.float32) for _ in range(depth)],
            *[
                pltpu.VMEM_SHARED((SMALL_VOCAB, DIM), jnp.float32)
                for _ in range(n_staged)
            ],
            pltpu.SemaphoreType.DMA,
            *[pltpu.SemaphoreType.DMA for _ in range(depth)],
            *[pltpu.SemaphoreType.DMA for _ in range(depth)],
        ],
    )
    def emb_kernel(*refs):
        feats = refs[:N_FIELDS]
        tables = refs[N_FIELDS : 2 * N_FIELDS]
        outs = refs[2 * N_FIELDS : 3 * N_FIELDS]
        scratch = refs[3 * N_FIELDS :]
        idx_v = scratch[0]
        rows = scratch[1 : 1 + depth]
        shared = scratch[1 + depth : 1 + depth + n_staged]
        isem = scratch[1 + depth + n_staged]
        gsems = scratch[2 + depth + n_staged : 2 + 2 * depth + n_staged]
        ssems = scratch[2 + 2 * depth + n_staged :]

        sid = lax.axis_index("s")
        wid = sid * nc + lax.axis_index("c")
        base = wid * bpw

        icps = [
            pltpu.async_copy(feats[i].at[pl.ds(base, bpw)], idx_v.at[i], isem)
            for i in range(N_FIELDS)
        ]

        for t in range(n_staged):

            @pl.when(sid == t)
            def _(t=t):
                pltpu.sync_copy(tables[staged_fields[t]], shared[t])

        for cp in icps:
            cp.wait()

        def fire_gather(pos):
            f, c = items[pos]
            b = pos % depth
            idx = idx_v.at[f, pl.ds(c * chunk, chunk)]
            src = (
                shared[f - N_BIG].at[idx]
                if f in staged_fields
                else tables[f].at[idx]
            )
            return pltpu.async_copy(src, rows[b], gsems[b])

        def fire_store(pos):
            f, c = items[pos]
            b = pos % depth
            return pltpu.async_copy(
                rows[b], outs[f].at[pl.ds(base + c * chunk, chunk)], ssems[b]
            )

        n_items = len(items)
        gcps = [None] * n_items
        scps = [None] * n_items
        for j in range(min(depth - 1, n_items)):
            gcps[j] = fire_gather(j)
        for i in range(n_items):
            if i >= 1:
                scps[i - 1].wait()
            j = i + depth - 1
            if j < n_items:
                if j == first_staged_item:
                    plsc.subcore_barrier()
                gcps[j] = fire_gather(j)
            gcps[i].wait()
            scps[i] = fire_store(i)
        scps[n_items - 1].wait()

    return emb_kernel


_emb_kernel = _build()


def kernel(
    feat_0, feat_1, feat_2, feat_3, feat_4, feat_5, feat_6, feat_7,
    feat_8, feat_9, feat_10, feat_11, feat_12, feat_13, feat_14, feat_15,
    feat_16, feat_17, feat_18, feat_19, feat_20, feat_21, feat_22, feat_23,
    feat_24, feat_25,
    W_0, W_1, W_2, W_3, W_4, W_5, W_6, W_7,
    W_8, W_9, W_10, W_11, W_12, W_13, W_14, W_15,
    W_16, W_17, W_18, W_19, W_20, W_21, W_22, W_23,
    W_24, W_25,
):
    args = locals()
    feats = [args[f"feat_{i}"] for i in range(N_FIELDS)]
    tables = [args[f"W_{i}"] for i in range(N_FIELDS)]
    outs = _emb_kernel(*feats, *tables)
    return tuple(o.reshape(BATCH, 1, DIM) for o in outs)


# speedup vs baseline: 1.0579x; 1.0023x over previous
import functools

import jax
import jax.numpy as jnp
from jax import lax
from jax.experimental import pallas as pl
from jax.experimental.pallas import tpu as pltpu
from jax.experimental.pallas import tpu_sc as plsc

DIM = 128
BATCH = 4096
N_FIELDS = 26
N_BIG = 6
SMALL_VOCAB = 1000


def _build():
    info = plsc.get_sparse_core_info()
    nc, ns = info.num_cores, info.num_subcores
    nw = nc * ns
    bpw = BATCH // nw

    depth = 5
    chunk = 128
    n_chunks = bpw // chunk
    n_staged = 5
    staged_fields = list(range(N_BIG, N_BIG + n_staged))
    hbm_fields = [f for f in range(N_FIELDS) if f not in staged_fields]
    order = hbm_fields + staged_fields
    items = [(f, c) for f in order for c in range(n_chunks)]
    first_staged_item = next(
        i for i, (f, _) in enumerate(items) if f in staged_fields
    )

    mesh = plsc.VectorSubcoreMesh(core_axis_name="c", subcore_axis_name="s")
    out_type = tuple(
        jax.ShapeDtypeStruct((BATCH, DIM), jnp.float32) for _ in range(N_FIELDS)
    )

    @functools.partial(
        pl.kernel,
        mesh=mesh,
        out_type=out_type,
        scratch_types=[
            pltpu.VMEM((N_FIELDS, bpw), jnp.int32),
            *[pltpu.VMEM((chunk, DIM), jnp.float32) for _ in range(depth)],
            *[
                pltpu.VMEM_SHARED((SMALL_VOCAB, DIM), jnp.float32)
                for _ in range(n_staged)
            ],
            pltpu.SemaphoreType.DMA,
            *[pltpu.SemaphoreType.DMA for _ in range(depth)],
            *[pltpu.SemaphoreType.DMA for _ in range(depth)],
        ],
    )
    def emb_kernel(*refs):
        feats = refs[:N_FIELDS]
        tables = refs[N_FIELDS : 2 * N_FIELDS]
        outs = refs[2 * N_FIELDS : 3 * N_FIELDS]
        scratch = refs[3 * N_FIELDS :]
        idx_v = scratch[0]
        rows = scratch[1 : 1 + depth]
        shared = scratch[1 + depth : 1 + depth + n_staged]
        isem = scratch[1 + depth + n_staged]
        gsems = scratch[2 + depth + n_staged : 2 + 2 * depth + n_staged]
        ssems = scratch[2 + 2 * depth + n_staged :]

        sid = lax.axis_index("s")
        wid = sid * nc + lax.axis_index("c")
        base = wid * bpw

        icps = [
            pltpu.async_copy(feats[i].at[pl.ds(base, bpw)], idx_v.at[i], isem)
            for i in range(N_FIELDS)
        ]

        for t in range(n_staged):

            @pl.when(sid == t)
            def _(t=t):
                pltpu.sync_copy(tables[staged_fields[t]], shared[t])

        for cp in icps:
            cp.wait()

        def fire_gather(pos):
            f, c = items[pos]
            b = pos % depth
            idx = idx_v.at[f, pl.ds(c * chunk, chunk)]
            src = (
                shared[f - N_BIG].at[idx]
                if f in staged_fields
                else tables[f].at[idx]
            )
            return pltpu.async_copy(src, rows[b], gsems[b])

        def fire_store(pos):
            f, c = items[pos]
            b = pos % depth
            return pltpu.async_copy(
                rows[b], outs[f].at[pl.ds(base + c * chunk, chunk)], ssems[b]
            )

        n_items = len(items)
        gcps = [None] * n_items
        scps = [None] * n_items
        for j in range(min(depth - 1, n_items)):
            gcps[j] = fire_gather(j)
        for i in range(n_items):
            if i >= 1:
                scps[i - 1].wait()
            j = i + depth - 1
            if j < n_items:
                if j == first_staged_item:
                    plsc.subcore_barrier()
                gcps[j] = fire_gather(j)
            gcps[i].wait()
            scps[i] = fire_store(i)
        scps[n_items - 1].wait()

    return emb_kernel


_emb_kernel = _build()


def kernel(
    feat_0, feat_1, feat_2, feat_3, feat_4, feat_5, feat_6, feat_7,
    feat_8, feat_9, feat_10, feat_11, feat_12, feat_13, feat_14, feat_15,
    feat_16, feat_17, feat_18, feat_19, feat_20, feat_21, feat_22, feat_23,
    feat_24, feat_25,
    W_0, W_1, W_2, W_3, W_4, W_5, W_6, W_7,
    W_8, W_9, W_10, W_11, W_12, W_13, W_14, W_15,
    W_16, W_17, W_18, W_19, W_20, W_21, W_22, W_23,
    W_24, W_25,
):
    args = locals()
    feats = [args[f"feat_{i}"] for i in range(N_FIELDS)]
    tables = [args[f"W_{i}"] for i in range(N_FIELDS)]
    outs = _emb_kernel(*feats, *tables)
    return tuple(o.reshape(BATCH, 1, DIM) for o in outs)
